# baseline (speedup 1.0000x reference)
import jax
import jax.numpy as jnp
from jax import lax
from jax.experimental import pallas as pl
from jax.experimental.pallas import tpu as pltpu
from jax.experimental.pallas import tpu_sc as plsc

VOCAB = 1000000
EMBED_DIM = 64
BATCH = 16384
TITLE_LEN = 20

NUM_CORES = 2
NUM_SUBCORES = 16
LANES = 16
NUM_WORKERS = NUM_CORES * NUM_SUBCORES

B_PER_W = BATCH // NUM_WORKERS
TOK_PER_W = B_PER_W * TITLE_LEN
IDX_ROW = 128
IDX_ROWS_PER_W = TOK_PER_W // IDX_ROW
CB = 32
TOK_PER_CHUNK = CB * TITLE_LEN
GATHERS_PER_CHUNK = TOK_PER_CHUNK // IDX_ROW
NSTEPS = B_PER_W // CB
D_SLICES = EMBED_DIM // LANES


def _body(title_hbm, table_hbm, out_hbm, idx_v, rows_v, out_v, gsem0, gsem1):
    wid = lax.axis_index("s") * NUM_CORES + lax.axis_index("c")

    pltpu.sync_copy(title_hbm.at[pl.ds(wid * IDX_ROWS_PER_W, IDX_ROWS_PER_W), :],
                    idx_v)

    gsems = (gsem0, gsem1)

    def start_chunk(s, buf):
        for g in range(GATHERS_PER_CHUNK):
            pltpu.async_copy(
                table_hbm.at[idx_v.at[s * GATHERS_PER_CHUNK + g]],
                rows_v.at[buf, pl.ds(g * IDX_ROW, IDX_ROW), :],
                gsems[buf])

    def wait_chunk(buf):
        pltpu.make_async_copy(
            table_hbm.at[idx_v.at[0]],
            rows_v.at[buf, pl.ds(0, IDX_ROW), :],
            gsems[buf]).wait()

    def reduce_chunk(s, buf):
        rows = rows_v.at[buf]
        inv_l = jnp.float32(1.0 / TITLE_LEN)

        def body(b, carry):
            tok = b * TITLE_LEN
            for d in range(D_SLICES):
                sl = pl.ds(d * LANES, LANES)
                vals = [rows[tok + t, sl] for t in range(TITLE_LEN)]
                while len(vals) > 1:
                    nxt = [vals[i] + vals[i + 1] for i in range(0, len(vals) - 1, 2)]
                    if len(vals) % 2:
                        nxt.append(vals[-1])
                    vals = nxt
                out_v[s * CB + b, sl] = vals[0] * inv_l
            return carry

        lax.fori_loop(0, CB, body, 0)

    start_chunk(0, 0)
    for s in range(NSTEPS):
        buf = s % 2
        if s + 1 < NSTEPS:
            start_chunk(s + 1, 1 - buf)
        for g in range(GATHERS_PER_CHUNK):
            wait_chunk(buf)
        reduce_chunk(s, buf)

    pltpu.sync_copy(out_v, out_hbm.at[pl.ds(wid * B_PER_W, B_PER_W), :])


@jax.jit
def kernel(title, word_emb_table):
    title2d = title.astype(jnp.int32).reshape(NUM_WORKERS * IDX_ROWS_PER_W, IDX_ROW)
    mesh = plsc.VectorSubcoreMesh(core_axis_name="c", subcore_axis_name="s")
    f = pl.kernel(
        _body,
        out_type=jax.ShapeDtypeStruct((BATCH, EMBED_DIM), jnp.float32),
        mesh=mesh,
        scratch_types=[
            pltpu.VMEM((IDX_ROWS_PER_W, IDX_ROW), jnp.int32),
            pltpu.VMEM((2, TOK_PER_CHUNK, EMBED_DIM), jnp.float32),
            pltpu.VMEM((B_PER_W, EMBED_DIM), jnp.float32),
            pltpu.SemaphoreType.DMA,
            pltpu.SemaphoreType.DMA,
        ],
        compiler_params=pltpu.CompilerParams(use_tc_tiling_on_sc=False),
    )
    return f(title2d, word_emb_table)
